# baseline (speedup 1.0000x reference)
import jax
import jax.numpy as jnp
from jax import lax
from jax.experimental import pallas as pl
from jax.experimental.pallas import tpu as pltpu
from jax.experimental.pallas import tpu_sc as plsc

B = 16384
F = 16
NC = 2
NS = 16
NW = NC * NS
BPW = B // NW
Q = 128
V = 1000000
TW = 32768
GR = (V + TW - 1) // TW


def _eye(rows):
    r = lax.broadcasted_iota(jnp.int32, (rows, 128), 0)
    c = lax.broadcasted_iota(jnp.int32, (rows, 128), 1)
    return jnp.where(r == c, jnp.float32(1), jnp.float32(0))


def _tc_body(gut_ref, dgut_ref, git_ref, dgit_ref, bi_ref, out_ref):
    s = jnp.concatenate(
        [gut_ref[...] + dgut_ref[...],
         git_ref[...] + dgit_ref[...],
         bi_ref[...].reshape(1, TW)], axis=0)
    out_ref[...] = lax.dot_general(
        s, _eye(2 * F + 1), (((0,), (0,)), ((), ())),
        preferred_element_type=jnp.float32)


def _sum_padded(gut, dgut, git, dgit, bi):
    return pl.pallas_call(
        _tc_body,
        grid=(GR,),
        in_specs=[
            pl.BlockSpec((F, TW), lambda i: (0, i)),
            pl.BlockSpec((F, TW), lambda i: (0, i)),
            pl.BlockSpec((F, TW), lambda i: (0, i)),
            pl.BlockSpec((F, TW), lambda i: (0, i)),
            pl.BlockSpec((TW,), lambda i: (i,)),
        ],
        out_specs=pl.BlockSpec((TW, 128), lambda i: (i, 0)),
        out_shape=jax.ShapeDtypeStruct((V, 128), jnp.float32),
        compiler_params=pltpu.CompilerParams(
            fuse_transposed_lhs_in_matmul=True),
    )(gut, dgut, git, dgit, bi)


def _sc_body(user_hbm, item_hbm, s_hbm,
             xui_out, beta_out, guo_out, gio_out,
             uidx_v, iidx_v, bufu, bufi, gu_st, gi_st, xui_v, beta_v,
             sem_u, sem_i):
    wid = lax.axis_index("s") * NC + lax.axis_index("c")
    base = wid * BPW
    lane = lax.iota(jnp.int32, F)

    for p in range(BPW // Q):
        pltpu.sync_copy(user_hbm.at[pl.ds(base + p * Q, Q)], uidx_v.at[p])
        pltpu.sync_copy(item_hbm.at[pl.ds(base + p * Q, Q)], iidx_v.at[p])

    for p in range(BPW // Q):
        cu = pltpu.async_copy(s_hbm.at[uidx_v.at[p]], bufu, sem_u)
        ci = pltpu.async_copy(s_hbm.at[iidx_v.at[p]], bufi, sem_i)
        cu.wait()
        ci.wait()

        def grp(g, carry, p=p):
            acc = jnp.zeros((F,), jnp.float32)
            bacc = jnp.zeros((F,), jnp.float32)
            for r in range(F):
                row = g * F + r
                u_vec = bufu[row, pl.ds(0, F)]
                i_vec = bufi[row, pl.ds(F, F)]
                b16 = bufi[row, pl.ds(2 * F, F)]
                b = b16[0]
                s = jnp.sum(u_vec * i_vec)
                acc = jnp.where(lane == r, s + b, acc)
                bacc = jnp.where(lane == r, b, bacc)
                st_row = p * 16 + 2 * g + (r // 8)
                gu_st[st_row, pl.ds((r % 8) * F, F)] = u_vec
                gi_st[st_row, pl.ds((r % 8) * F, F)] = i_vec
            xui_v[pl.ds(p * Q + g * F, F)] = acc
            beta_v[pl.ds(p * Q + g * F, F)] = bacc
            return carry

        lax.fori_loop(0, Q // F, grp, 0)

    pltpu.sync_copy(gu_st, guo_out.at[pl.ds(wid * 64, 64), :])
    pltpu.sync_copy(gi_st, gio_out.at[pl.ds(wid * 64, 64), :])
    pltpu.sync_copy(xui_v, xui_out.at[pl.ds(base, BPW)])
    pltpu.sync_copy(beta_v, beta_out.at[pl.ds(base, BPW)])


def _gather_dot(user, item, S):
    f = pl.kernel(
        _sc_body,
        out_type=(
            jax.ShapeDtypeStruct((B,), jnp.float32),
            jax.ShapeDtypeStruct((B,), jnp.float32),
            jax.ShapeDtypeStruct((B * F // 128, 128), jnp.float32),
            jax.ShapeDtypeStruct((B * F // 128, 128), jnp.float32),
        ),
        mesh=plsc.VectorSubcoreMesh(core_axis_name="c", subcore_axis_name="s"),
        compiler_params=pltpu.CompilerParams(needs_layout_passes=False),
        scratch_types=[
            pltpu.VMEM((BPW // Q, Q), jnp.int32),
            pltpu.VMEM((BPW // Q, Q), jnp.int32),
            pltpu.VMEM((Q, 128), jnp.float32),
            pltpu.VMEM((Q, 128), jnp.float32),
            pltpu.VMEM((64, 128), jnp.float32),
            pltpu.VMEM((64, 128), jnp.float32),
            pltpu.VMEM((BPW,), jnp.float32),
            pltpu.VMEM((BPW,), jnp.float32),
            pltpu.SemaphoreType.DMA,
            pltpu.SemaphoreType.DMA,
        ],
    )
    return f(user, item, S)


@jax.jit
def _run(user, item, Bi, GuT, GiT, Delta_GuT, Delta_GiT):
    s = _sum_padded(GuT, Delta_GuT, GiT, Delta_GiT, Bi)
    xui, beta_i, guo, gio = _gather_dot(user, item, s)
    return xui, beta_i, guo.reshape(B, F), gio.reshape(B, F)


def kernel(user, item, Bi, Gu, Gi, Delta_Gu, Delta_Gi):
    return _run(user, item, Bi, Gu.T, Gi.T, Delta_Gu.T, Delta_Gi.T)
